# baseline (speedup 1.0000x reference)
import functools

import jax
import jax.numpy as jnp
from jax import lax
from jax.experimental import pallas as pl
from jax.experimental.pallas import tpu as pltpu
from jax.experimental.pallas import tpu_sc as plsc

IN_F = 16384
OUT_F = 16384
K = 64

NC = 2
NS = 16
NW = NC * NS
CHUNK = 128
ROWS_PER_SUB = IN_F // NS


def _make_sc_spmm(num_chunks):
    mesh = plsc.VectorSubcoreMesh(core_axis_name="c", subcore_axis_name="s")

    @functools.partial(
        pl.kernel,
        out_type=jax.ShapeDtypeStruct((IN_F, NC * K), jnp.float32),
        mesh=mesh,
        scratch_types=[
            pltpu.VMEM((CHUNK, K), jnp.bfloat16),
            pltpu.VMEM((CHUNK, K), jnp.bfloat16),
            pltpu.VMEM((CHUNK, K), jnp.float32),
            pltpu.VMEM((CHUNK, K), jnp.float32),
            [pltpu.VMEM((CHUNK,), jnp.int32)] * 4,
            [pltpu.VMEM((CHUNK,), jnp.int32)] * 4,
            [pltpu.VMEM((CHUNK,), jnp.float32)] * 4,
            pltpu.VMEM_SHARED((IN_F, K), jnp.bfloat16),
            pltpu.VMEM_SHARED((IN_F, K), jnp.float32),
            [pltpu.SemaphoreType.DMA] * 2,
            [pltpu.SemaphoreType.DMA] * 2,
            [pltpu.SemaphoreType.DMA] * 4,
            [pltpu.SemaphoreType.DMA] * 4,
            [pltpu.SemaphoreType.DMA] * 4,
            pltpu.SemaphoreType.DMA,
        ],
        compiler_params=pltpu.CompilerParams(use_tc_tiling_on_sc=False,
                                             needs_layout_passes=False),
    )
    def sc_spmm(xp_hbm, rows_hbm, cols_hbm, vals_hbm, out_hbm,
                g0, g1, d0, d1, colsS, rowsS, valsS, xs, acc,
                gsems, ssems, csems, rsems, vsems, xsem):
        gaths = (g0, g1)
        dsts = (d0, d1)
        NR = 4

        c = lax.axis_index("c")
        s = lax.axis_index("s")
        wid = c * NS + s

        def idx_slab(hbm, k):
            return hbm.at[pl.ds((wid * num_chunks + k) * CHUNK, CHUNK)]

        with jax.named_scope("stage_x"):
            xslab = pl.ds(s * ROWS_PER_SUB, ROWS_PER_SUB)
            pltpu.async_copy(xp_hbm.at[xslab], xs.at[xslab], xsem)
            for k0 in range(2):
                pltpu.async_copy(idx_slab(cols_hbm, k0), colsS[k0], csems[k0])
                pltpu.async_copy(idx_slab(rows_hbm, k0), rowsS[k0], rsems[k0])
                pltpu.async_copy(idx_slab(vals_hbm, k0), valsS[k0], vsems[k0])

        zero = jnp.zeros((16,), jnp.float32)

        def zbody(n, carry):
            for j in range(K // 16):
                d0[n, pl.ds(j * 16, 16)] = zero
            return carry

        with jax.named_scope("zero_acc"):
            lax.fori_loop(0, CHUNK, zbody, 0)
            for t in range(ROWS_PER_SUB // CHUNK):
                pltpu.sync_copy(d0, acc.at[pl.ds(s * ROWS_PER_SUB + t * CHUNK, CHUNK)])
            pltpu.make_async_copy(xp_hbm.at[xslab], xs.at[xslab], xsem).wait()
            plsc.subcore_barrier()

        pltpu.make_async_copy(idx_slab(cols_hbm, 0), colsS[0], csems[0]).wait()
        pltpu.async_copy(xs.at[colsS[0]], gaths[0], gsems[0])

        def scale(gath, vals_ref, dst):
            @plsc.parallel_loop(0, CHUNK // 16, unroll=2)
            def gbody(g):
                vvec = vals_ref[pl.ds(g * 16, 16)]
                for i in range(16):
                    v = vvec[i]
                    n = g * 16 + i
                    for h in range(K // 32):
                        packed = gath[n, pl.ds(h * 32, 32)]
                        a, b2 = plsc.unpack(
                            packed, format=plsc.PackFormat.INTERLEAVED)
                        dst[n, pl.ds(h * 32, 16)] = a * v
                        dst[n, pl.ds(h * 32 + 16, 16)] = b2 * v

        def group_body(p, carry):
            for b in range(NR):
                k = p * NR + b
                bb = b % 2
                bbn = 1 - bb
                s1 = (b + 1) % NR
                s6 = (b + 2) % NR

                pltpu.make_async_copy(xs.at[colsS[b]], gaths[bb],
                                      gsems[bb]).wait()

                @pl.when(k + 1 < num_chunks)
                def _():
                    pltpu.make_async_copy(idx_slab(cols_hbm, k + 1),
                                          colsS[s1], csems[s1]).wait()
                    pltpu.async_copy(xs.at[colsS[s1]], gaths[bbn],
                                     gsems[bbn])

                @pl.when(k >= 2)
                def _():
                    pltpu.make_async_copy(dsts[bb], acc.at[rowsS[s6]],
                                          ssems[bb]).wait()

                @pl.when(k + 2 < num_chunks)
                def _():
                    pltpu.async_copy(idx_slab(cols_hbm, k + 2), colsS[s6],
                                     csems[s6])
                    pltpu.async_copy(idx_slab(rows_hbm, k + 2), rowsS[s6],
                                     rsems[s6])
                    pltpu.async_copy(idx_slab(vals_hbm, k + 2), valsS[s6],
                                     vsems[s6])

                pltpu.make_async_copy(idx_slab(vals_hbm, k), valsS[b],
                                      vsems[b]).wait()
                scale(gaths[bb], valsS[b], dsts[bb])

                pltpu.make_async_copy(idx_slab(rows_hbm, k), rowsS[b],
                                      rsems[b]).wait()
                pltpu.async_copy(dsts[bb], acc.at[rowsS[b]], ssems[bb],
                                 add=True)
            return carry

        with jax.named_scope("main_loop"):
            lax.fori_loop(0, num_chunks // NR, group_body, 0)

            for k in range(num_chunks - 2, num_chunks):
                pltpu.make_async_copy(dsts[k % 2], acc.at[rowsS[k % NR]],
                                      ssems[k % 2]).wait()
            plsc.subcore_barrier()

        with jax.named_scope("writeback"):
            for t in range(ROWS_PER_SUB // CHUNK):
                off = s * ROWS_PER_SUB + t * CHUNK
                pltpu.sync_copy(acc.at[pl.ds(off, CHUNK)],
                                out_hbm.at[pl.ds(off, CHUNK),
                                           pl.ds(c * K, K)])

    return sc_spmm


def _combine_body(p_ref, b_ref, o_ref):
    v = p_ref[...]
    o_ref[...] = v[:, :K] + v[:, K:] + b_ref[...]


@jax.jit
def kernel(x, rows, cols, vals, bias):
    nnz = rows.shape[0]
    num_chunks = -(-nnz // (NW * CHUNK))
    num_chunks = -(-num_chunks // 4) * 4
    padded = NW * num_chunks * CHUNK
    pad = padded - nnz

    rows_p = jnp.pad(rows.astype(jnp.int32), (0, pad))
    cols_p = jnp.pad(cols.astype(jnp.int32), (0, pad))
    vals_p = jnp.pad(vals, (0, pad))

    xp = (x.astype(jnp.bfloat16)
          .reshape(IN_F, 2, 2, 16)
          .transpose(0, 1, 3, 2)
          .reshape(IN_F, K))

    partial = _make_sc_spmm(num_chunks)(xp, rows_p, cols_p, vals_p)

    out = pl.pallas_call(
        _combine_body,
        out_shape=jax.ShapeDtypeStruct((IN_F, K), jnp.float32),
    )(partial, bias.reshape(IN_F, 1))
    return out


# speedup vs baseline: 1.0457x; 1.0457x over previous
import functools

import jax
import jax.numpy as jnp
from jax import lax
from jax.experimental import pallas as pl
from jax.experimental.pallas import tpu as pltpu
from jax.experimental.pallas import tpu_sc as plsc

IN_F = 16384
OUT_F = 16384
K = 64

NC = 2
NS = 16
NW = NC * NS
CHUNK = 128
ROWS_PER_SUB = IN_F // NS


def _make_sc_spmm(num_chunks):
    mesh = plsc.VectorSubcoreMesh(core_axis_name="c", subcore_axis_name="s")

    @functools.partial(
        pl.kernel,
        out_type=jax.ShapeDtypeStruct((IN_F, NC * K), jnp.float32),
        mesh=mesh,
        scratch_types=[
            pltpu.VMEM((CHUNK, K), jnp.bfloat16),
            pltpu.VMEM((CHUNK, K), jnp.bfloat16),
            pltpu.VMEM((CHUNK, K), jnp.float32),
            pltpu.VMEM((CHUNK, K), jnp.float32),
            [pltpu.VMEM((CHUNK,), jnp.int32)] * 4,
            [pltpu.VMEM((CHUNK,), jnp.int32)] * 4,
            [pltpu.VMEM((CHUNK,), jnp.float32)] * 4,
            pltpu.VMEM_SHARED((IN_F, K), jnp.bfloat16),
            pltpu.VMEM_SHARED((IN_F, K), jnp.float32),
            [pltpu.SemaphoreType.DMA] * 2,
            [pltpu.SemaphoreType.DMA] * 2,
            [pltpu.SemaphoreType.DMA] * 4,
            [pltpu.SemaphoreType.DMA] * 4,
            [pltpu.SemaphoreType.DMA] * 4,
            pltpu.SemaphoreType.DMA,
        ],
        compiler_params=pltpu.CompilerParams(use_tc_tiling_on_sc=False,
                                             needs_layout_passes=False),
    )
    def sc_spmm(xp_hbm, rows_hbm, cols_hbm, vals_hbm, out_hbm,
                g0, g1, d0, d1, colsS, rowsS, valsS, xs, acc,
                gsems, ssems, csems, rsems, vsems, xsem):
        gaths = (g0, g1)
        dsts = (d0, d1)
        NR = 4

        c = lax.axis_index("c")
        s = lax.axis_index("s")
        wid = c * NS + s

        def idx_slab(hbm, k):
            return hbm.at[pl.ds((wid * num_chunks + k) * CHUNK, CHUNK)]

        with jax.named_scope("stage_x"):
            xslab = pl.ds(s * ROWS_PER_SUB, ROWS_PER_SUB)
            pltpu.async_copy(xp_hbm.at[xslab], xs.at[xslab], xsem)
            for k0 in range(2):
                pltpu.async_copy(idx_slab(cols_hbm, k0), colsS[k0], csems[k0])
                pltpu.async_copy(idx_slab(rows_hbm, k0), rowsS[k0], rsems[k0])
                pltpu.async_copy(idx_slab(vals_hbm, k0), valsS[k0], vsems[k0])

        zero = jnp.zeros((16,), jnp.float32)

        def zbody(n, carry):
            for j in range(K // 16):
                d0[n, pl.ds(j * 16, 16)] = zero
            return carry

        with jax.named_scope("zero_acc"):
            lax.fori_loop(0, CHUNK, zbody, 0)
            for t in range(ROWS_PER_SUB // CHUNK):
                pltpu.sync_copy(d0, acc.at[pl.ds(s * ROWS_PER_SUB + t * CHUNK, CHUNK)])
            pltpu.make_async_copy(xp_hbm.at[xslab], xs.at[xslab], xsem).wait()
            plsc.subcore_barrier()

        pltpu.make_async_copy(idx_slab(cols_hbm, 0), colsS[0], csems[0]).wait()
        pltpu.async_copy(xs.at[colsS[0]], gaths[0], gsems[0])

        def scale(gath, vals_ref, dst):
            @plsc.parallel_loop(0, CHUNK // 16, unroll=2)
            def gbody(g):
                vvec = vals_ref[pl.ds(g * 16, 16)]
                for i in range(16):
                    v = vvec[i]
                    n = g * 16 + i
                    for h in range(K // 32):
                        packed = gath[n, pl.ds(h * 32, 32)]
                        a, b2 = plsc.unpack(
                            packed, format=plsc.PackFormat.INTERLEAVED)
                        dst[n, pl.ds(h * 32, 16)] = a * v
                        dst[n, pl.ds(h * 32 + 16, 16)] = b2 * v

        def group_body(p, carry):
            for b in range(NR):
                k = p * NR + b
                bb = b % 2
                bbn = 1 - bb
                s1 = (b + 1) % NR
                s6 = (b + 2) % NR

                pltpu.make_async_copy(xs.at[colsS[b]], gaths[bb],
                                      gsems[bb]).wait()

                @pl.when(k + 1 < num_chunks)
                def _():
                    pltpu.make_async_copy(idx_slab(cols_hbm, k + 1),
                                          colsS[s1], csems[s1]).wait()
                    pltpu.async_copy(xs.at[colsS[s1]], gaths[bbn],
                                     gsems[bbn])

                @pl.when(k >= 2)
                def _():
                    pltpu.make_async_copy(dsts[bb], acc.at[rowsS[s6]],
                                          ssems[bb]).wait()

                @pl.when(k + 2 < num_chunks)
                def _():
                    pltpu.async_copy(idx_slab(cols_hbm, k + 2), colsS[s6],
                                     csems[s6])
                    pltpu.async_copy(idx_slab(rows_hbm, k + 2), rowsS[s6],
                                     rsems[s6])
                    pltpu.async_copy(idx_slab(vals_hbm, k + 2), valsS[s6],
                                     vsems[s6])

                pltpu.make_async_copy(idx_slab(vals_hbm, k), valsS[b],
                                      vsems[b]).wait()
                scale(gaths[bb], valsS[b], dsts[bb])

                pltpu.make_async_copy(idx_slab(rows_hbm, k), rowsS[b],
                                      rsems[b]).wait()
                pltpu.async_copy(dsts[bb], acc.at[rowsS[b]], ssems[bb],
                                 add=True)
            return carry

        with jax.named_scope("main_loop"):
            lax.fori_loop(0, num_chunks // NR, group_body, 0)

            for k in range(num_chunks - 2, num_chunks):
                pltpu.make_async_copy(dsts[k % 2], acc.at[rowsS[k % NR]],
                                      ssems[k % 2]).wait()
            plsc.subcore_barrier()

        with jax.named_scope("writeback"):
            for t in range(ROWS_PER_SUB // CHUNK):
                off = s * ROWS_PER_SUB + t * CHUNK
                pltpu.sync_copy(acc.at[pl.ds(off, CHUNK)],
                                out_hbm.at[pl.ds(off, CHUNK),
                                           pl.ds(c * K, K)])

    return sc_spmm


def _combine_body(p_ref, perm_ref, b_ref, o_ref):
    v = p_ref[...]
    s = v[:, :K] + v[:, K:]
    o_ref[...] = jnp.dot(s, perm_ref[...],
                         preferred_element_type=jnp.float32) + b_ref[...]


def _perm_matrix():
    import numpy as np
    p = np.zeros((K, K), dtype=np.float32)
    for d in range(K):
        h, i = divmod(d, 32)
        orig = h * 32 + (2 * i if i < 16 else 2 * (i - 16) + 1)
        p[d, orig] = 1.0
    return jnp.asarray(p)


@jax.jit
def kernel(x, rows, cols, vals, bias):
    nnz = rows.shape[0]
    num_chunks = -(-nnz // (NW * CHUNK))
    num_chunks = -(-num_chunks // 4) * 4
    padded = NW * num_chunks * CHUNK
    pad = padded - nnz

    rows_p = jnp.pad(rows.astype(jnp.int32), (0, pad))
    cols_p = jnp.pad(cols.astype(jnp.int32), (0, pad))
    vals_p = jnp.pad(vals, (0, pad))

    xp = x.astype(jnp.bfloat16)

    partial = _make_sc_spmm(num_chunks)(xp, rows_p, cols_p, vals_p)

    R = 2048
    out = pl.pallas_call(
        _combine_body,
        out_shape=jax.ShapeDtypeStruct((IN_F, K), jnp.float32),
        grid=(IN_F // R,),
        in_specs=[
            pl.BlockSpec((R, NC * K), lambda i: (i, 0)),
            pl.BlockSpec((K, K), lambda i: (0, 0)),
            pl.BlockSpec((R, 1), lambda i: (i, 0)),
        ],
        out_specs=pl.BlockSpec((R, K), lambda i: (i, 0)),
    )(partial, _perm_matrix(), bias.reshape(IN_F, 1))
    return out
